# Initial kernel scaffold; baseline (speedup 1.0000x reference)
import jax
import jax.numpy as jnp
from jax.experimental import pallas as pl


def kernel(h, edge_index, d, w_real, w_imag, t1r_W, t1r_b, t1i_W, t1i_b, W1_0, b1_0, W2_0, b2_0, W1_1, b1_1, W2_1, b2_1, t2_W, t2_b):
    raise NotImplementedError("write your pallas kernel here")


# speedup vs baseline: 1.2190x; 1.2190x over previous
import functools
import jax
import jax.numpy as jnp
from jax import lax
from jax.experimental import pallas as pl
from jax.experimental.pallas import tpu as pltpu

N_NODES = 10000
HIDDEN = 128
HALF = 64
BN = 1000


def _dot_t(x, w):
    return lax.dot_general(x, w, (((1,), (1,)), ((), ())),
                           preferred_element_type=jnp.float32)


def _pre_body(h_ref, wr_ref, br_ref, wi_ref, bi_ref, h0_ref, h1_ref):
    h = h_ref[...]
    hr = jax.nn.relu(_dot_t(h, wr_ref[...]) + br_ref[...])
    hi = jax.nn.relu(_dot_t(h, wi_ref[...]) + bi_ref[...])
    h0_ref[...] = jnp.concatenate([hr[:, :HALF], hi[:, :HALF]], axis=1)
    h1_ref[...] = jnp.concatenate([hr[:, HALF:], hi[:, HALF:]], axis=1)


def _tc_pre(h, t1r_W, t1r_b, t1i_W, t1i_b):
    grid = (N_NODES // BN,)
    return pl.pallas_call(
        _pre_body,
        grid=grid,
        in_specs=[
            pl.BlockSpec((BN, HIDDEN), lambda i: (i, 0)),
            pl.BlockSpec((HIDDEN, HIDDEN), lambda i: (0, 0)),
            pl.BlockSpec((1, HIDDEN), lambda i: (0, 0)),
            pl.BlockSpec((HIDDEN, HIDDEN), lambda i: (0, 0)),
            pl.BlockSpec((1, HIDDEN), lambda i: (0, 0)),
        ],
        out_specs=[
            pl.BlockSpec((BN, HIDDEN), lambda i: (i, 0)),
            pl.BlockSpec((BN, HIDDEN), lambda i: (i, 0)),
        ],
        out_shape=[
            jax.ShapeDtypeStruct((N_NODES, HIDDEN), jnp.float32),
            jax.ShapeDtypeStruct((N_NODES, HIDDEN), jnp.float32),
        ],
    )(h, t1r_W, t1r_b.reshape(1, HIDDEN), t1i_W, t1i_b.reshape(1, HIDDEN))


def _assemble_z(zp_ref):
    z0 = zp_ref[0, 0] + zp_ref[1, 0]
    z1 = zp_ref[0, 1] + zp_ref[1, 1]
    zr = jnp.concatenate([z0[:, :HALF], z1[:, :HALF]], axis=1)
    zi = jnp.concatenate([z0[:, HALF:], z1[:, HALF:]], axis=1)
    return zr, zi


def _complex_linear(zr, zi, w1_ref, b1_ref, w2_ref, b2_ref):
    zr_new = (_dot_t(zr, w1_ref[...]) + b1_ref[...]) - (_dot_t(zi, w2_ref[...]) + b2_ref[...])
    zi_new = (_dot_t(zr_new, w2_ref[...]) + b2_ref[...]) + (_dot_t(zi, w1_ref[...]) + b1_ref[...])
    return jax.nn.relu(zr_new), jax.nn.relu(zi_new)


def _mid_body(zp_ref, w1_ref, b1_ref, w2_ref, b2_ref, h0_ref, h1_ref):
    zr, zi = _assemble_z(zp_ref)
    hr, hi = _complex_linear(zr, zi, w1_ref, b1_ref, w2_ref, b2_ref)
    h0_ref[...] = jnp.concatenate([hr[:, :HALF], hi[:, :HALF]], axis=1)
    h1_ref[...] = jnp.concatenate([hr[:, HALF:], hi[:, HALF:]], axis=1)


def _tc_mid(zp, W1, b1, W2, b2):
    grid = (N_NODES // BN,)
    return pl.pallas_call(
        _mid_body,
        grid=grid,
        in_specs=[
            pl.BlockSpec((2, 2, BN, HIDDEN), lambda i: (0, 0, i, 0)),
            pl.BlockSpec((HIDDEN, HIDDEN), lambda i: (0, 0)),
            pl.BlockSpec((1, HIDDEN), lambda i: (0, 0)),
            pl.BlockSpec((HIDDEN, HIDDEN), lambda i: (0, 0)),
            pl.BlockSpec((1, HIDDEN), lambda i: (0, 0)),
        ],
        out_specs=[
            pl.BlockSpec((BN, HIDDEN), lambda i: (i, 0)),
            pl.BlockSpec((BN, HIDDEN), lambda i: (i, 0)),
        ],
        out_shape=[
            jax.ShapeDtypeStruct((N_NODES, HIDDEN), jnp.float32),
            jax.ShapeDtypeStruct((N_NODES, HIDDEN), jnp.float32),
        ],
    )(zp, W1, b1.reshape(1, HIDDEN), W2, b2.reshape(1, HIDDEN))


def _post_body(zp_ref, w1_ref, b1_ref, w2_ref, b2_ref, t2w_ref, t2b_ref, out_ref):
    zr, zi = _assemble_z(zp_ref)
    hr, hi = _complex_linear(zr, zi, w1_ref, b1_ref, w2_ref, b2_ref)
    t2w = t2w_ref[...]
    logits = (_dot_t(hr, t2w[:, :HIDDEN]) + _dot_t(hi, t2w[:, HIDDEN:])) + t2b_ref[...]
    m = jnp.max(logits, axis=1, keepdims=True)
    s = logits - m
    lse = jnp.log(jnp.sum(jnp.exp(s), axis=1, keepdims=True))
    out_ref[...] = s - lse


def _tc_post(zp, W1, b1, W2, b2, t2_W, t2_b):
    grid = (N_NODES // BN,)
    out_dim = t2_W.shape[0]
    return pl.pallas_call(
        _post_body,
        grid=grid,
        in_specs=[
            pl.BlockSpec((2, 2, BN, HIDDEN), lambda i: (0, 0, i, 0)),
            pl.BlockSpec((HIDDEN, HIDDEN), lambda i: (0, 0)),
            pl.BlockSpec((1, HIDDEN), lambda i: (0, 0)),
            pl.BlockSpec((HIDDEN, HIDDEN), lambda i: (0, 0)),
            pl.BlockSpec((1, HIDDEN), lambda i: (0, 0)),
            pl.BlockSpec((out_dim, 2 * HIDDEN), lambda i: (0, 0)),
            pl.BlockSpec((1, out_dim), lambda i: (0, 0)),
        ],
        out_specs=[pl.BlockSpec((BN, out_dim), lambda i: (i, 0))],
        out_shape=[jax.ShapeDtypeStruct((N_NODES, out_dim), jnp.float32)],
    )(zp, W1, b1.reshape(1, HIDDEN), W2, b2.reshape(1, HIDDEN),
      t2_W, t2_b.reshape(1, out_dim))[0]


def _sparse_layer_jnp(h0, h1, src, dst, er, ei):
    n = N_NODES
    zp_rows = []
    for hp in (h0, h1):
        rows = hp[src]
        hr = rows[:, :HALF]
        hi = rows[:, HALF:]
        zr = hr * er[:, None] - hi * ei[:, None]
        zi = hr * ei[:, None] + hi * er[:, None]
        contrib = jnp.concatenate([zr, zi], axis=1)
        zp_rows.append(jax.ops.segment_sum(contrib, dst, num_segments=n))
    z = jnp.stack(zp_rows)
    zp = jnp.stack([z, jnp.zeros_like(z)])
    return zp


def kernel(h, edge_index, d, w_real, w_imag, t1r_W, t1r_b, t1i_W, t1i_b,
           W1_0, b1_0, W2_0, b2_0, W1_1, b1_1, W2_1, b2_1, t2_W, t2_b):
    src = edge_index[0]
    dst = edge_index[1]
    dv = d[:, 0]
    coef = dv[dst] * dv[src]
    er = coef * w_real[:, 0]
    ei = coef * w_imag[:, 0]

    h0, h1 = _tc_pre(h, t1r_W, t1r_b, t1i_W, t1i_b)
    zp = _sparse_layer_jnp(h0, h1, src, dst, er, ei)
    h0, h1 = _tc_mid(zp, W1_0, b1_0, W2_0, b2_0)
    zp = _sparse_layer_jnp(h0, h1, src, dst, er, ei)
    return _tc_post(zp, W1_1, b1_1, W2_1, b2_1, t2_W, t2_b)


# speedup vs baseline: 8.0061x; 6.5680x over previous
import functools
import jax
import jax.numpy as jnp
from jax import lax
from jax.experimental import pallas as pl
from jax.experimental.pallas import tpu as pltpu
from jax.experimental.pallas import tpu_sc as plsc

N_NODES = 10000
N_EDGES = 320000
HIDDEN = 128
HALF = 64
BN = 1000
BNP = 1024

CH = 80
NCH = 125
TILES = 32
ROWS2D = N_EDGES // CH
NPAD = 10240
RPT = N_NODES // 16

_MESH = plsc.VectorSubcoreMesh(core_axis_name="c", subcore_axis_name="s")


def _dot_t(x, w):
    return lax.dot_general(x, w, (((1,), (1,)), ((), ())),
                           preferred_element_type=jnp.float32)


def _pre_body(h_ref, wr_ref, br_ref, wi_ref, bi_ref, h0_ref, h1_ref):
    h = h_ref[...]
    hr = jax.nn.relu(_dot_t(h, wr_ref[...]) + br_ref[...])
    hi = jax.nn.relu(_dot_t(h, wi_ref[...]) + bi_ref[...])
    h0_ref[...] = jnp.concatenate([hr[:, :HALF], hi[:, :HALF]], axis=1)
    h1_ref[...] = jnp.concatenate([hr[:, HALF:], hi[:, HALF:]], axis=1)


def _tc_pre(h, t1r_W, t1r_b, t1i_W, t1i_b):
    grid = (N_NODES // BN,)
    return pl.pallas_call(
        _pre_body,
        grid=grid,
        in_specs=[
            pl.BlockSpec((BN, HIDDEN), lambda i: (i, 0)),
            pl.BlockSpec((HIDDEN, HIDDEN), lambda i: (0, 0)),
            pl.BlockSpec((1, HIDDEN), lambda i: (0, 0)),
            pl.BlockSpec((HIDDEN, HIDDEN), lambda i: (0, 0)),
            pl.BlockSpec((1, HIDDEN), lambda i: (0, 0)),
        ],
        out_specs=[
            pl.BlockSpec((BN, HIDDEN), lambda i: (i, 0)),
            pl.BlockSpec((BN, HIDDEN), lambda i: (i, 0)),
        ],
        out_shape=[
            jax.ShapeDtypeStruct((N_NODES, HIDDEN), jnp.float32),
            jax.ShapeDtypeStruct((N_NODES, HIDDEN), jnp.float32),
        ],
    )(h, t1r_W, t1r_b.reshape(1, HIDDEN), t1i_W, t1i_b.reshape(1, HIDDEN))


def _assemble_z(zp_ref):
    z0 = zp_ref[0, 0] + zp_ref[1, 0]
    z1 = zp_ref[0, 1] + zp_ref[1, 1]
    zr = jnp.concatenate([z0[:, :HALF], z1[:, :HALF]], axis=1)
    zi = jnp.concatenate([z0[:, HALF:], z1[:, HALF:]], axis=1)
    return zr, zi


def _complex_linear(zr, zi, w1_ref, b1_ref, w2_ref, b2_ref):
    zr_new = (_dot_t(zr, w1_ref[...]) + b1_ref[...]) - (_dot_t(zi, w2_ref[...]) + b2_ref[...])
    zi_new = (_dot_t(zr_new, w2_ref[...]) + b2_ref[...]) + (_dot_t(zi, w1_ref[...]) + b1_ref[...])
    return jax.nn.relu(zr_new), jax.nn.relu(zi_new)


def _mid_body(zp_ref, w1_ref, b1_ref, w2_ref, b2_ref, h0_ref, h1_ref):
    zr, zi = _assemble_z(zp_ref)
    hr, hi = _complex_linear(zr, zi, w1_ref, b1_ref, w2_ref, b2_ref)
    h0_ref[...] = jnp.concatenate([hr[:, :HALF], hi[:, :HALF]], axis=1)
    h1_ref[...] = jnp.concatenate([hr[:, HALF:], hi[:, HALF:]], axis=1)


def _tc_mid(zp, W1, b1, W2, b2):
    grid = (NPAD // BNP,)
    return pl.pallas_call(
        _mid_body,
        grid=grid,
        in_specs=[
            pl.BlockSpec((2, 2, BNP, HIDDEN), lambda i: (0, 0, i, 0)),
            pl.BlockSpec((HIDDEN, HIDDEN), lambda i: (0, 0)),
            pl.BlockSpec((1, HIDDEN), lambda i: (0, 0)),
            pl.BlockSpec((HIDDEN, HIDDEN), lambda i: (0, 0)),
            pl.BlockSpec((1, HIDDEN), lambda i: (0, 0)),
        ],
        out_specs=[
            pl.BlockSpec((BNP, HIDDEN), lambda i: (i, 0)),
            pl.BlockSpec((BNP, HIDDEN), lambda i: (i, 0)),
        ],
        out_shape=[
            jax.ShapeDtypeStruct((NPAD, HIDDEN), jnp.float32),
            jax.ShapeDtypeStruct((NPAD, HIDDEN), jnp.float32),
        ],
    )(zp, W1, b1.reshape(1, HIDDEN), W2, b2.reshape(1, HIDDEN))


def _post_body(zp_ref, w1_ref, b1_ref, w2_ref, b2_ref, t2w_ref, t2b_ref, out_ref):
    zr, zi = _assemble_z(zp_ref)
    hr, hi = _complex_linear(zr, zi, w1_ref, b1_ref, w2_ref, b2_ref)
    t2w = t2w_ref[...]
    logits = (_dot_t(hr, t2w[:, :HIDDEN]) + _dot_t(hi, t2w[:, HIDDEN:])) + t2b_ref[...]
    m = jnp.max(logits, axis=1, keepdims=True)
    sh = logits - m
    lse = jnp.log(jnp.sum(jnp.exp(sh), axis=1, keepdims=True))
    out_ref[...] = sh - lse


def _tc_post(zp, W1, b1, W2, b2, t2_W, t2_b):
    grid = (NPAD // BNP,)
    out_dim = t2_W.shape[0]
    return pl.pallas_call(
        _post_body,
        grid=grid,
        in_specs=[
            pl.BlockSpec((2, 2, BNP, HIDDEN), lambda i: (0, 0, i, 0)),
            pl.BlockSpec((HIDDEN, HIDDEN), lambda i: (0, 0)),
            pl.BlockSpec((1, HIDDEN), lambda i: (0, 0)),
            pl.BlockSpec((HIDDEN, HIDDEN), lambda i: (0, 0)),
            pl.BlockSpec((1, HIDDEN), lambda i: (0, 0)),
            pl.BlockSpec((out_dim, 2 * HIDDEN), lambda i: (0, 0)),
            pl.BlockSpec((1, out_dim), lambda i: (0, 0)),
        ],
        out_specs=[pl.BlockSpec((BNP, out_dim), lambda i: (i, 0))],
        out_shape=[jax.ShapeDtypeStruct((NPAD, out_dim), jnp.float32)],
    )(zp, W1, b1.reshape(1, HIDDEN), W2, b2.reshape(1, HIDDEN),
      t2_W, t2_b.reshape(1, out_dim))[0]


def _coef_body(d_hbm, src_hbm, dst_hbm, wr_hbm, wi_hbm, er_out, ei_out,
               sbuf, tbuf, wrbuf, wibuf, erb, eib, dsv, ddv, sem1, sem2):
    c = lax.axis_index("c")
    s = lax.axis_index("s")
    w = c * 16 + s
    pltpu.sync_copy(src_hbm.at[w], sbuf)
    pltpu.sync_copy(dst_hbm.at[w], tbuf)
    pltpu.sync_copy(wr_hbm.at[w], wrbuf)
    pltpu.sync_copy(wi_hbm.at[w], wibuf)

    def chunk(i, _):
        cp1 = pltpu.async_copy(d_hbm.at[sbuf.at[i]], dsv, sem1)
        cp2 = pltpu.async_copy(d_hbm.at[tbuf.at[i]], ddv, sem2)
        cp1.wait()
        cp2.wait()
        for g in range(CH // 16):
            sl = pl.ds(g * 16, 16)
            cf = dsv[sl] * ddv[sl]
            erb[i, sl] = cf * wrbuf[i, sl]
            eib[i, sl] = cf * wibuf[i, sl]
        return 0

    lax.fori_loop(0, NCH, chunk, 0)
    pltpu.sync_copy(erb, er_out.at[w])
    pltpu.sync_copy(eib, ei_out.at[w])


_sc_coef = functools.partial(
    pl.kernel,
    _coef_body,
    out_type=(jax.ShapeDtypeStruct((TILES, NCH, CH), jnp.float32),
              jax.ShapeDtypeStruct((TILES, NCH, CH), jnp.float32)),
    mesh=_MESH,
    scratch_types=[
        pltpu.VMEM((NCH, CH), jnp.int32),
        pltpu.VMEM((NCH, CH), jnp.int32),
        pltpu.VMEM((NCH, CH), jnp.float32),
        pltpu.VMEM((NCH, CH), jnp.float32),
        pltpu.VMEM((NCH, CH), jnp.float32),
        pltpu.VMEM((NCH, CH), jnp.float32),
        pltpu.VMEM((CH,), jnp.float32),
        pltpu.VMEM((CH,), jnp.float32),
        pltpu.SemaphoreType.DMA,
        pltpu.SemaphoreType.DMA,
    ],
)()


def _layer_body(h0_hbm, h1_hbm, meta_hbm, coef_hbm, zp_out, metab, coefb, hbuf, cbuf, zacc, gsem):
    c = lax.axis_index("c")
    s = lax.axis_index("s")
    w = c * 16 + s

    for p, hp in ((0, h0_hbm), (1, h1_hbm)):
        @plsc.parallel_loop(0, CH, 1)
        def zrow(r):
            for g in range(8):
                cbuf[r, pl.ds(g * 16, 16)] = jnp.zeros((16,), jnp.float32)

        for k in range(640 // CH):
            pltpu.sync_copy(cbuf, zacc.at[pl.ds(s * 640 + k * CH, CH)])
        plsc.subcore_barrier()

        def chunk(i, _):
            pltpu.sync_copy(meta_hbm.at[w, i], metab)
            pltpu.sync_copy(coef_hbm.at[w, i], coefb)
            pltpu.async_copy(hp.at[metab.at[0]], hbuf, gsem).wait()

            @plsc.parallel_loop(0, CH, 16)
            def egroup(j0):
                er16 = coefb[0, pl.ds(j0, 16)]
                ei16 = coefb[1, pl.ds(j0, 16)]
                for jj in range(16):
                    erv = jnp.broadcast_to(er16[jj], (16,))
                    eiv = jnp.broadcast_to(ei16[jj], (16,))
                    j = j0 + jj
                    for g in range(4):
                        slr = pl.ds(g * 16, 16)
                        sli = pl.ds(HALF + g * 16, 16)
                        hr = hbuf[j, slr]
                        hi = hbuf[j, sli]
                        cbuf[j, slr] = hr * erv - hi * eiv
                        cbuf[j, sli] = hr * eiv + hi * erv

            pltpu.sync_copy(cbuf, zacc.at[metab.at[1]], add=True)
            return 0

        lax.fori_loop(0, NCH, chunk, 0)
        plsc.subcore_barrier()
        pltpu.sync_copy(zacc.at[pl.ds(s * 640, 640)],
                        zp_out.at[c, p, pl.ds(s * 640, 640)])
        plsc.subcore_barrier()


_sc_layer = functools.partial(
    pl.kernel,
    _layer_body,
    out_type=jax.ShapeDtypeStruct((2, 2, NPAD, HIDDEN), jnp.float32),
    mesh=_MESH,
    scratch_types=[
        pltpu.VMEM((2, CH), jnp.int32),
        pltpu.VMEM((2, CH), jnp.float32),
        pltpu.VMEM((CH, HIDDEN), jnp.float32),
        pltpu.VMEM((CH, HIDDEN), jnp.float32),
        pltpu.VMEM_SHARED((NPAD, HIDDEN), jnp.float32),
        pltpu.SemaphoreType.DMA,
    ],
)()


def kernel(h, edge_index, d, w_real, w_imag, t1r_W, t1r_b, t1i_W, t1i_b,
           W1_0, b1_0, W2_0, b2_0, W1_1, b1_1, W2_1, b2_1, t2_W, t2_b):
    src3d = edge_index[0].reshape(TILES, NCH, CH)
    dst3d = edge_index[1].reshape(TILES, NCH, CH)
    wr3d = w_real.reshape(TILES, NCH, CH)
    wi3d = w_imag.reshape(TILES, NCH, CH)
    d1 = d.reshape(N_NODES)

    er3d, ei3d = _sc_coef(d1, src3d, dst3d, wr3d, wi3d)
    meta = jnp.stack([src3d, dst3d], axis=2)
    coef = jnp.stack([er3d, ei3d], axis=2)
    h0, h1 = _tc_pre(h, t1r_W, t1r_b, t1i_W, t1i_b)
    zp = _sc_layer(h0, h1, meta, coef)
    h0, h1 = _tc_mid(zp, W1_0, b1_0, W2_0, b2_0)
    zp = _sc_layer(h0, h1, meta, coef)
    return _tc_post(zp, W1_1, b1_1, W2_1, b2_1, t2_W, t2_b)[:N_NODES]


# speedup vs baseline: 17.0181x; 2.1256x over previous
import functools
import jax
import jax.numpy as jnp
from jax import lax
from jax.experimental import pallas as pl
from jax.experimental.pallas import tpu as pltpu
from jax.experimental.pallas import tpu_sc as plsc

N_NODES = 10000
N_EDGES = 320000
HIDDEN = 128
HALF = 64
BN = 1000
BNP = 1024

CH = 80
NCH = 125
NCHP = 127
TILES = 32
NPAD = 10240
SBYTES = CH * HIDDEN * 4

_MESH = plsc.VectorSubcoreMesh(core_axis_name="c", subcore_axis_name="s")


def _dot_t(x, w):
    return lax.dot_general(x, w, (((1,), (1,)), ((), ())),
                           preferred_element_type=jnp.float32)


def _pre_body(h_ref, wr_ref, br_ref, wi_ref, bi_ref, h0_ref, h1_ref):
    h = h_ref[...]
    hr = jax.nn.relu(_dot_t(h, wr_ref[...]) + br_ref[...])
    hi = jax.nn.relu(_dot_t(h, wi_ref[...]) + bi_ref[...])
    h0_ref[...] = jnp.concatenate([hr[:, :HALF], hi[:, :HALF]], axis=1)
    h1_ref[...] = jnp.concatenate([hr[:, HALF:], hi[:, HALF:]], axis=1)


def _tc_pre(h, t1r_W, t1r_b, t1i_W, t1i_b):
    grid = (N_NODES // BN,)
    return pl.pallas_call(
        _pre_body,
        grid=grid,
        in_specs=[
            pl.BlockSpec((BN, HIDDEN), lambda i: (i, 0)),
            pl.BlockSpec((HIDDEN, HIDDEN), lambda i: (0, 0)),
            pl.BlockSpec((1, HIDDEN), lambda i: (0, 0)),
            pl.BlockSpec((HIDDEN, HIDDEN), lambda i: (0, 0)),
            pl.BlockSpec((1, HIDDEN), lambda i: (0, 0)),
        ],
        out_specs=[
            pl.BlockSpec((BN, HIDDEN), lambda i: (i, 0)),
            pl.BlockSpec((BN, HIDDEN), lambda i: (i, 0)),
        ],
        out_shape=[
            jax.ShapeDtypeStruct((N_NODES, HIDDEN), jnp.float32),
            jax.ShapeDtypeStruct((N_NODES, HIDDEN), jnp.float32),
        ],
    )(h, t1r_W, t1r_b.reshape(1, HIDDEN), t1i_W, t1i_b.reshape(1, HIDDEN))


def _assemble_z(zp_ref):
    z0 = zp_ref[0, 0] + zp_ref[1, 0]
    z1 = zp_ref[0, 1] + zp_ref[1, 1]
    zr = jnp.concatenate([z0[:, :HALF], z1[:, :HALF]], axis=1)
    zi = jnp.concatenate([z0[:, HALF:], z1[:, HALF:]], axis=1)
    return zr, zi


def _complex_linear(zr, zi, w1_ref, b1_ref, w2_ref, b2_ref):
    zr_new = (_dot_t(zr, w1_ref[...]) + b1_ref[...]) - (_dot_t(zi, w2_ref[...]) + b2_ref[...])
    zi_new = (_dot_t(zr_new, w2_ref[...]) + b2_ref[...]) + (_dot_t(zi, w1_ref[...]) + b1_ref[...])
    return jax.nn.relu(zr_new), jax.nn.relu(zi_new)


def _mid_body(zp_ref, w1_ref, b1_ref, w2_ref, b2_ref, h0_ref, h1_ref):
    zr, zi = _assemble_z(zp_ref)
    hr, hi = _complex_linear(zr, zi, w1_ref, b1_ref, w2_ref, b2_ref)
    h0_ref[...] = jnp.concatenate([hr[:, :HALF], hi[:, :HALF]], axis=1)
    h1_ref[...] = jnp.concatenate([hr[:, HALF:], hi[:, HALF:]], axis=1)


def _tc_mid(zp, W1, b1, W2, b2):
    grid = (NPAD // BNP,)
    return pl.pallas_call(
        _mid_body,
        grid=grid,
        in_specs=[
            pl.BlockSpec((2, 2, BNP, HIDDEN), lambda i: (0, 0, i, 0)),
            pl.BlockSpec((HIDDEN, HIDDEN), lambda i: (0, 0)),
            pl.BlockSpec((1, HIDDEN), lambda i: (0, 0)),
            pl.BlockSpec((HIDDEN, HIDDEN), lambda i: (0, 0)),
            pl.BlockSpec((1, HIDDEN), lambda i: (0, 0)),
        ],
        out_specs=[
            pl.BlockSpec((BNP, HIDDEN), lambda i: (i, 0)),
            pl.BlockSpec((BNP, HIDDEN), lambda i: (i, 0)),
        ],
        out_shape=[
            jax.ShapeDtypeStruct((NPAD, HIDDEN), jnp.float32),
            jax.ShapeDtypeStruct((NPAD, HIDDEN), jnp.float32),
        ],
    )(zp, W1, b1.reshape(1, HIDDEN), W2, b2.reshape(1, HIDDEN))


def _post_body(zp_ref, w1_ref, b1_ref, w2_ref, b2_ref, t2w_ref, t2b_ref, out_ref):
    zr, zi = _assemble_z(zp_ref)
    hr, hi = _complex_linear(zr, zi, w1_ref, b1_ref, w2_ref, b2_ref)
    t2w = t2w_ref[...]
    logits = (_dot_t(hr, t2w[:, :HIDDEN]) + _dot_t(hi, t2w[:, HIDDEN:])) + t2b_ref[...]
    m = jnp.max(logits, axis=1, keepdims=True)
    sh = logits - m
    lse = jnp.log(jnp.sum(jnp.exp(sh), axis=1, keepdims=True))
    out_ref[...] = sh - lse


def _tc_post(zp, W1, b1, W2, b2, t2_W, t2_b):
    grid = (NPAD // BNP,)
    out_dim = t2_W.shape[0]
    return pl.pallas_call(
        _post_body,
        grid=grid,
        in_specs=[
            pl.BlockSpec((2, 2, BNP, HIDDEN), lambda i: (0, 0, i, 0)),
            pl.BlockSpec((HIDDEN, HIDDEN), lambda i: (0, 0)),
            pl.BlockSpec((1, HIDDEN), lambda i: (0, 0)),
            pl.BlockSpec((HIDDEN, HIDDEN), lambda i: (0, 0)),
            pl.BlockSpec((1, HIDDEN), lambda i: (0, 0)),
            pl.BlockSpec((out_dim, 2 * HIDDEN), lambda i: (0, 0)),
            pl.BlockSpec((1, out_dim), lambda i: (0, 0)),
        ],
        out_specs=[pl.BlockSpec((BNP, out_dim), lambda i: (i, 0))],
        out_shape=[jax.ShapeDtypeStruct((NPAD, out_dim), jnp.float32)],
    )(zp, W1, b1.reshape(1, HIDDEN), W2, b2.reshape(1, HIDDEN),
      t2_W, t2_b.reshape(1, out_dim))[0]


def _coef_body(d_hbm, src_hbm, dst_hbm, wr_hbm, wi_hbm, er_out, ei_out,
               sbuf, tbuf, wrbuf, wibuf, erb, eib, dsv, ddv, sem1, sem2):
    c = lax.axis_index("c")
    s = lax.axis_index("s")
    w = c * 16 + s
    pltpu.sync_copy(src_hbm.at[w], sbuf)
    pltpu.sync_copy(dst_hbm.at[w], tbuf)
    pltpu.sync_copy(wr_hbm.at[w], wrbuf)
    pltpu.sync_copy(wi_hbm.at[w], wibuf)

    pltpu.async_copy(d_hbm.at[sbuf.at[0]], dsv.at[0], sem1)
    pltpu.async_copy(d_hbm.at[tbuf.at[0]], ddv.at[0], sem2)

    def pair(ii, _):
        for b in (0, 1):
            i = 2 * ii + b
            nb = 1 - b
            pltpu.async_copy(d_hbm.at[sbuf.at[i + 1]], dsv.at[nb], sem1)
            pltpu.async_copy(d_hbm.at[tbuf.at[i + 1]], ddv.at[nb], sem2)
            pltpu.make_async_copy(d_hbm.at[sbuf.at[i]], dsv.at[b], sem1).wait()
            pltpu.make_async_copy(d_hbm.at[tbuf.at[i]], ddv.at[b], sem2).wait()
            for g in range(CH // 16):
                sl = pl.ds(g * 16, 16)
                cf = dsv[b, sl] * ddv[b, sl]
                erb[i, sl] = cf * wrbuf[i, sl]
                eib[i, sl] = cf * wibuf[i, sl]
        return 0

    lax.fori_loop(0, (NCH - 1) // 2, pair, 0)
    i = NCH - 1
    pltpu.make_async_copy(d_hbm.at[sbuf.at[i]], dsv.at[0], sem1).wait()
    pltpu.make_async_copy(d_hbm.at[tbuf.at[i]], ddv.at[0], sem2).wait()
    for g in range(CH // 16):
        sl = pl.ds(g * 16, 16)
        cf = dsv[0, sl] * ddv[0, sl]
        erb[i, sl] = cf * wrbuf[i, sl]
        eib[i, sl] = cf * wibuf[i, sl]
    pltpu.sync_copy(erb, er_out.at[w])
    pltpu.sync_copy(eib, ei_out.at[w])


_sc_coef = functools.partial(
    pl.kernel,
    _coef_body,
    out_type=(jax.ShapeDtypeStruct((TILES, NCH, CH), jnp.float32),
              jax.ShapeDtypeStruct((TILES, NCH, CH), jnp.float32)),
    mesh=_MESH,
    scratch_types=[
        pltpu.VMEM((NCH, CH), jnp.int32),
        pltpu.VMEM((NCH, CH), jnp.int32),
        pltpu.VMEM((NCH, CH), jnp.float32),
        pltpu.VMEM((NCH, CH), jnp.float32),
        pltpu.VMEM((NCH, CH), jnp.float32),
        pltpu.VMEM((NCH, CH), jnp.float32),
        pltpu.VMEM((2, CH), jnp.float32),
        pltpu.VMEM((2, CH), jnp.float32),
        pltpu.SemaphoreType.DMA,
        pltpu.SemaphoreType.DMA,
    ],
)()


def _scale_chunk(coefb_b, hbuf_b):
    @plsc.parallel_loop(0, CH, 16)
    def egroup(j0):
        er16 = coefb_b[0, pl.ds(j0, 16)]
        ei16 = coefb_b[1, pl.ds(j0, 16)]
        for jj in range(16):
            erv = jnp.broadcast_to(er16[jj], (16,))
            eiv = jnp.broadcast_to(ei16[jj], (16,))
            j = j0 + jj
            for g in range(HALF // 16):
                slr = pl.ds(g * 16, 16)
                sli = pl.ds(HALF + g * 16, 16)
                hr = hbuf_b[j, slr]
                hi = hbuf_b[j, sli]
                hbuf_b[j, slr] = hr * erv - hi * eiv
                hbuf_b[j, sli] = hr * eiv + hi * erv


def _layer_body(h0_hbm, h1_hbm, midx_hbm, coef_hbm, zp_out,
                midx0, midx1, coefb0, coefb1, sidx, didx0, didx1,
                hbuf0, hbuf1, zacc,
                gsem0, gsem1, ssem0, ssem1, msem0, msem1, csem0, csem1):
    c = lax.axis_index("c")
    s = lax.axis_index("s")
    w = c * 16 + s
    midxs = (midx0, midx1)
    coefs = (coefb0, coefb1)
    didxs = (didx0, didx1)
    hbufs = (hbuf0, hbuf1)
    gsems = (gsem0, gsem1)
    ssems = (ssem0, ssem1)
    msems = (msem0, msem1)
    csems = (csem0, csem1)

    def unpack(b):
        for g in range(CH // 16):
            sl = pl.ds(g * 16, 16)
            v = midxs[b][sl]
            sidx[sl] = jnp.bitwise_and(v, 0xFFFF)
            didxs[b][sl] = jnp.right_shift(v, 16)

    def missue(cc, b):
        pltpu.async_copy(midx_hbm.at[w, cc], midxs[b], msems[b])

    def mwait(b):
        pltpu.make_async_copy(midx_hbm.at[w, 0], midxs[b], msems[b]).wait()

    def cissue(cc, b):
        pltpu.async_copy(coef_hbm.at[w, cc], coefs[b], csems[b])

    def cwait(b):
        pltpu.make_async_copy(coef_hbm.at[w, 0], coefs[b], csems[b]).wait()

    def gissue(hp, b):
        pltpu.async_copy(hp.at[sidx], hbufs[b], gsems[b])

    def gwait(hp, b):
        pltpu.make_async_copy(hp.at[sidx], hbufs[b], gsems[b]).wait()

    def sissue(b):
        pltpu.async_copy(hbufs[b], zacc.at[didxs[b]], ssems[b], add=True)

    def swait(b):
        pltpu.make_async_copy(hbufs[b], zacc.at[didxs[b]], ssems[b]).wait()

    for p, hp in ((0, h0_hbm), (1, h1_hbm)):
        @plsc.parallel_loop(0, CH, 1)
        def zrow(r):
            for g in range(HIDDEN // 16):
                hbuf0[r, pl.ds(g * 16, 16)] = jnp.zeros((16,), jnp.float32)
                hbuf1[r, pl.ds(g * 16, 16)] = jnp.zeros((16,), jnp.float32)

        for g in range(CH // 16):
            didx1[pl.ds(g * 16, 16)] = jnp.zeros((16,), jnp.int32)

        for k in range(640 // CH):
            pltpu.sync_copy(hbuf0, zacc.at[pl.ds(s * 640 + k * CH, CH)])
        plsc.subcore_barrier()

        sissue(1)
        missue(0, 0)
        mwait(0)
        unpack(0)
        gissue(hp, 0)
        missue(1, 1)
        missue(2, 0)
        cissue(0, 0)
        cissue(1, 1)

        def pair(ii, _):
            for b in (0, 1):
                cc = 2 * ii + b
                nb = 1 - b
                gwait(hp, b)
                swait(nb)
                mwait(nb)
                unpack(nb)
                gissue(hp, nb)
                missue(cc + 3, nb)
                cwait(b)
                _scale_chunk(coefs[b], hbufs[b])
                cissue(cc + 2, b)
                sissue(b)
            return 0

        lax.fori_loop(0, (NCH - 1) // 2, pair, 0)
        gwait(hp, 0)
        swait(1)
        cwait(0)
        _scale_chunk(coefb0, hbuf0)
        sissue(0)
        swait(0)
        mwait(1)
        mwait(0)
        cwait(1)

        plsc.subcore_barrier()
        pltpu.sync_copy(zacc.at[pl.ds(s * 640, 640)],
                        zp_out.at[c, p, pl.ds(s * 640, 640)])
        plsc.subcore_barrier()


_sc_layer = functools.partial(
    pl.kernel,
    _layer_body,
    out_type=jax.ShapeDtypeStruct((2, 2, NPAD, HIDDEN), jnp.float32),
    mesh=_MESH,
    scratch_types=[
        pltpu.VMEM((CH,), jnp.int32),
        pltpu.VMEM((CH,), jnp.int32),
        pltpu.VMEM((2, CH), jnp.float32),
        pltpu.VMEM((2, CH), jnp.float32),
        pltpu.VMEM((CH,), jnp.int32),
        pltpu.VMEM((CH,), jnp.int32),
        pltpu.VMEM((CH,), jnp.int32),
        pltpu.VMEM((CH, HIDDEN), jnp.float32),
        pltpu.VMEM((CH, HIDDEN), jnp.float32),
        pltpu.VMEM_SHARED((NPAD, HIDDEN), jnp.float32),
        pltpu.SemaphoreType.DMA,
        pltpu.SemaphoreType.DMA,
        pltpu.SemaphoreType.DMA,
        pltpu.SemaphoreType.DMA,
        pltpu.SemaphoreType.DMA,
        pltpu.SemaphoreType.DMA,
        pltpu.SemaphoreType.DMA,
        pltpu.SemaphoreType.DMA,
    ],
)()


def kernel(h, edge_index, d, w_real, w_imag, t1r_W, t1r_b, t1i_W, t1i_b,
           W1_0, b1_0, W2_0, b2_0, W1_1, b1_1, W2_1, b2_1, t2_W, t2_b):
    src3d = edge_index[0].reshape(TILES, NCH, CH)
    dst3d = edge_index[1].reshape(TILES, NCH, CH)
    wr3d = w_real.reshape(TILES, NCH, CH)
    wi3d = w_imag.reshape(TILES, NCH, CH)
    d1 = d.reshape(N_NODES)

    er3d, ei3d = _sc_coef(d1, src3d, dst3d, wr3d, wi3d)
    packed = jnp.bitwise_or(src3d, jnp.left_shift(dst3d, 16))
    midx = jnp.pad(packed, ((0, 0), (0, NCHP - NCH), (0, 0)))
    coef = jnp.pad(jnp.stack([er3d, ei3d], axis=2),
                   ((0, 0), (0, NCHP - NCH), (0, 0), (0, 0)))
    h0, h1 = _tc_pre(h, t1r_W, t1r_b, t1i_W, t1i_b)
    zp = _sc_layer(h0, h1, midx, coef)
    h0, h1 = _tc_mid(zp, W1_0, b1_0, W2_0, b2_0)
    zp = _sc_layer(h0, h1, midx, coef)
    return _tc_post(zp, W1_1, b1_1, W2_1, b2_1, t2_W, t2_b)[:N_NODES]


# speedup vs baseline: 17.2233x; 1.0121x over previous
import functools
import jax
import jax.numpy as jnp
from jax import lax
from jax.experimental import pallas as pl
from jax.experimental.pallas import tpu as pltpu
from jax.experimental.pallas import tpu_sc as plsc

N_NODES = 10000
N_EDGES = 320000
HIDDEN = 128
HALF = 64
BN = 1000
BNP = 1024

CH = 80
NCH = 125
NCHP = 127
TILES = 32
NPAD = 10240
SBYTES = CH * HIDDEN * 4

_MESH = plsc.VectorSubcoreMesh(core_axis_name="c", subcore_axis_name="s")


def _dot_t(x, w):
    return lax.dot_general(x, w, (((1,), (1,)), ((), ())),
                           preferred_element_type=jnp.float32)


def _pre_body(h_ref, wr_ref, br_ref, wi_ref, bi_ref, h0_ref, h1_ref):
    h = h_ref[...]
    hr = jax.nn.relu(_dot_t(h, wr_ref[...]) + br_ref[...])
    hi = jax.nn.relu(_dot_t(h, wi_ref[...]) + bi_ref[...])
    h0_ref[...] = jnp.concatenate([hr[:, :HALF], hi[:, :HALF]], axis=1)
    h1_ref[...] = jnp.concatenate([hr[:, HALF:], hi[:, HALF:]], axis=1)


def _tc_pre(h, t1r_W, t1r_b, t1i_W, t1i_b):
    grid = (N_NODES // BN,)
    return pl.pallas_call(
        _pre_body,
        grid=grid,
        in_specs=[
            pl.BlockSpec((BN, HIDDEN), lambda i: (i, 0)),
            pl.BlockSpec((HIDDEN, HIDDEN), lambda i: (0, 0)),
            pl.BlockSpec((1, HIDDEN), lambda i: (0, 0)),
            pl.BlockSpec((HIDDEN, HIDDEN), lambda i: (0, 0)),
            pl.BlockSpec((1, HIDDEN), lambda i: (0, 0)),
        ],
        out_specs=[
            pl.BlockSpec((BN, HIDDEN), lambda i: (i, 0)),
            pl.BlockSpec((BN, HIDDEN), lambda i: (i, 0)),
        ],
        out_shape=[
            jax.ShapeDtypeStruct((N_NODES, HIDDEN), jnp.float32),
            jax.ShapeDtypeStruct((N_NODES, HIDDEN), jnp.float32),
        ],
    )(h, t1r_W, t1r_b.reshape(1, HIDDEN), t1i_W, t1i_b.reshape(1, HIDDEN))


def _assemble_z(zp_ref):
    z0 = zp_ref[0, 0] + zp_ref[1, 0]
    z1 = zp_ref[0, 1] + zp_ref[1, 1]
    zr = jnp.concatenate([z0[:, :HALF], z1[:, :HALF]], axis=1)
    zi = jnp.concatenate([z0[:, HALF:], z1[:, HALF:]], axis=1)
    return zr, zi


def _complex_linear(zr, zi, w1_ref, b1_ref, w2_ref, b2_ref):
    zr_new = (_dot_t(zr, w1_ref[...]) + b1_ref[...]) - (_dot_t(zi, w2_ref[...]) + b2_ref[...])
    zi_new = (_dot_t(zr_new, w2_ref[...]) + b2_ref[...]) + (_dot_t(zi, w1_ref[...]) + b1_ref[...])
    return jax.nn.relu(zr_new), jax.nn.relu(zi_new)


def _mid_body(zp_ref, w1_ref, b1_ref, w2_ref, b2_ref, h0_ref, h1_ref):
    zr, zi = _assemble_z(zp_ref)
    hr, hi = _complex_linear(zr, zi, w1_ref, b1_ref, w2_ref, b2_ref)
    h0_ref[...] = jnp.concatenate([hr[:, :HALF], hi[:, :HALF]], axis=1)
    h1_ref[...] = jnp.concatenate([hr[:, HALF:], hi[:, HALF:]], axis=1)


def _tc_mid(zp, W1, b1, W2, b2):
    grid = (NPAD // BNP,)
    return pl.pallas_call(
        _mid_body,
        grid=grid,
        in_specs=[
            pl.BlockSpec((2, 2, BNP, HIDDEN), lambda i: (0, 0, i, 0)),
            pl.BlockSpec((HIDDEN, HIDDEN), lambda i: (0, 0)),
            pl.BlockSpec((1, HIDDEN), lambda i: (0, 0)),
            pl.BlockSpec((HIDDEN, HIDDEN), lambda i: (0, 0)),
            pl.BlockSpec((1, HIDDEN), lambda i: (0, 0)),
        ],
        out_specs=[
            pl.BlockSpec((BNP, HIDDEN), lambda i: (i, 0)),
            pl.BlockSpec((BNP, HIDDEN), lambda i: (i, 0)),
        ],
        out_shape=[
            jax.ShapeDtypeStruct((NPAD, HIDDEN), jnp.float32),
            jax.ShapeDtypeStruct((NPAD, HIDDEN), jnp.float32),
        ],
    )(zp, W1, b1.reshape(1, HIDDEN), W2, b2.reshape(1, HIDDEN))


def _post_body(zp_ref, w1_ref, b1_ref, w2_ref, b2_ref, t2w_ref, t2b_ref, out_ref):
    zr, zi = _assemble_z(zp_ref)
    hr, hi = _complex_linear(zr, zi, w1_ref, b1_ref, w2_ref, b2_ref)
    t2w = t2w_ref[...]
    logits = (_dot_t(hr, t2w[:, :HIDDEN]) + _dot_t(hi, t2w[:, HIDDEN:])) + t2b_ref[...]
    m = jnp.max(logits, axis=1, keepdims=True)
    sh = logits - m
    lse = jnp.log(jnp.sum(jnp.exp(sh), axis=1, keepdims=True))
    out_ref[...] = sh - lse


def _tc_post(zp, W1, b1, W2, b2, t2_W, t2_b):
    grid = (NPAD // BNP,)
    out_dim = t2_W.shape[0]
    return pl.pallas_call(
        _post_body,
        grid=grid,
        in_specs=[
            pl.BlockSpec((2, 2, BNP, HIDDEN), lambda i: (0, 0, i, 0)),
            pl.BlockSpec((HIDDEN, HIDDEN), lambda i: (0, 0)),
            pl.BlockSpec((1, HIDDEN), lambda i: (0, 0)),
            pl.BlockSpec((HIDDEN, HIDDEN), lambda i: (0, 0)),
            pl.BlockSpec((1, HIDDEN), lambda i: (0, 0)),
            pl.BlockSpec((out_dim, 2 * HIDDEN), lambda i: (0, 0)),
            pl.BlockSpec((1, out_dim), lambda i: (0, 0)),
        ],
        out_specs=[pl.BlockSpec((BNP, out_dim), lambda i: (i, 0))],
        out_shape=[jax.ShapeDtypeStruct((NPAD, out_dim), jnp.float32)],
    )(zp, W1, b1.reshape(1, HIDDEN), W2, b2.reshape(1, HIDDEN),
      t2_W, t2_b.reshape(1, out_dim))[0]


def _coef_body(d_hbm, src_hbm, dst_hbm, wr_hbm, wi_hbm, midx_out, coef_out,
               sbuf, tbuf, wrbuf, wibuf, mpk, cfb, dsv, ddv, sem1, sem2):
    c = lax.axis_index("c")
    s = lax.axis_index("s")
    w = c * 16 + s
    pltpu.sync_copy(src_hbm.at[w], sbuf)
    pltpu.sync_copy(dst_hbm.at[w], tbuf)
    pltpu.sync_copy(wr_hbm.at[w], wrbuf)
    pltpu.sync_copy(wi_hbm.at[w], wibuf)

    pltpu.async_copy(d_hbm.at[sbuf.at[0]], dsv.at[0], sem1)
    pltpu.async_copy(d_hbm.at[tbuf.at[0]], ddv.at[0], sem2)

    def combine(i, b):
        for g in range(CH // 16):
            sl = pl.ds(g * 16, 16)
            sv = sbuf[i, sl]
            tv = tbuf[i, sl]
            mpk[i, sl] = jnp.bitwise_or(sv, jnp.left_shift(tv, 16))
            cf = dsv[b, sl] * ddv[b, sl]
            cfb[i, 0, sl] = cf * wrbuf[i, sl]
            cfb[i, 1, sl] = cf * wibuf[i, sl]

    def pair(ii, _):
        for b in (0, 1):
            i = 2 * ii + b
            nb = 1 - b
            pltpu.async_copy(d_hbm.at[sbuf.at[i + 1]], dsv.at[nb], sem1)
            pltpu.async_copy(d_hbm.at[tbuf.at[i + 1]], ddv.at[nb], sem2)
            pltpu.make_async_copy(d_hbm.at[sbuf.at[i]], dsv.at[b], sem1).wait()
            pltpu.make_async_copy(d_hbm.at[tbuf.at[i]], ddv.at[b], sem2).wait()
            combine(i, b)
        return 0

    lax.fori_loop(0, (NCH - 1) // 2, pair, 0)
    i = NCH - 1
    pltpu.make_async_copy(d_hbm.at[sbuf.at[i]], dsv.at[0], sem1).wait()
    pltpu.make_async_copy(d_hbm.at[tbuf.at[i]], ddv.at[0], sem2).wait()
    combine(i, 0)
    pltpu.sync_copy(mpk, midx_out.at[w])
    pltpu.sync_copy(cfb, coef_out.at[w])


_sc_coef = functools.partial(
    pl.kernel,
    _coef_body,
    out_type=(jax.ShapeDtypeStruct((TILES, NCHP, CH), jnp.int32),
              jax.ShapeDtypeStruct((TILES, NCHP, 2, CH), jnp.float32)),
    mesh=_MESH,
    scratch_types=[
        pltpu.VMEM((NCH, CH), jnp.int32),
        pltpu.VMEM((NCH, CH), jnp.int32),
        pltpu.VMEM((NCH, CH), jnp.float32),
        pltpu.VMEM((NCH, CH), jnp.float32),
        pltpu.VMEM((NCHP, CH), jnp.int32),
        pltpu.VMEM((NCHP, 2, CH), jnp.float32),
        pltpu.VMEM((2, CH), jnp.float32),
        pltpu.VMEM((2, CH), jnp.float32),
        pltpu.SemaphoreType.DMA,
        pltpu.SemaphoreType.DMA,
    ],
)()


def _scale_chunk(coefb_b, hbuf_b):
    @plsc.parallel_loop(0, CH, 16)
    def egroup(j0):
        er16 = coefb_b[0, pl.ds(j0, 16)]
        ei16 = coefb_b[1, pl.ds(j0, 16)]
        for jj in range(16):
            erv = jnp.broadcast_to(er16[jj], (16,))
            eiv = jnp.broadcast_to(ei16[jj], (16,))
            j = j0 + jj
            for g in range(HALF // 16):
                slr = pl.ds(g * 16, 16)
                sli = pl.ds(HALF + g * 16, 16)
                hr = hbuf_b[j, slr]
                hi = hbuf_b[j, sli]
                hbuf_b[j, slr] = hr * erv - hi * eiv
                hbuf_b[j, sli] = hr * eiv + hi * erv


def _layer_body(h0_hbm, h1_hbm, midx_hbm, coef_hbm, zp_out,
                midx0, midx1, coefb0, coefb1, sidx, didx0, didx1,
                hbuf0, hbuf1, zacc,
                gsem0, gsem1, ssem0, ssem1, msem0, msem1, csem0, csem1):
    c = lax.axis_index("c")
    s = lax.axis_index("s")
    w = c * 16 + s
    midxs = (midx0, midx1)
    coefs = (coefb0, coefb1)
    didxs = (didx0, didx1)
    hbufs = (hbuf0, hbuf1)
    gsems = (gsem0, gsem1)
    ssems = (ssem0, ssem1)
    msems = (msem0, msem1)
    csems = (csem0, csem1)

    def unpack(b):
        for g in range(CH // 16):
            sl = pl.ds(g * 16, 16)
            v = midxs[b][sl]
            sidx[sl] = jnp.bitwise_and(v, 0xFFFF)
            didxs[b][sl] = jnp.right_shift(v, 16)

    def missue(cc, b):
        pltpu.async_copy(midx_hbm.at[w, cc], midxs[b], msems[b])

    def mwait(b):
        pltpu.make_async_copy(midx_hbm.at[w, 0], midxs[b], msems[b]).wait()

    def cissue(cc, b):
        pltpu.async_copy(coef_hbm.at[w, cc], coefs[b], csems[b])

    def cwait(b):
        pltpu.make_async_copy(coef_hbm.at[w, 0], coefs[b], csems[b]).wait()

    def gissue(hp, b):
        pltpu.async_copy(hp.at[sidx], hbufs[b], gsems[b])

    def gwait(hp, b):
        pltpu.make_async_copy(hp.at[sidx], hbufs[b], gsems[b]).wait()

    def sissue(b):
        pltpu.async_copy(hbufs[b], zacc.at[didxs[b]], ssems[b], add=True)

    def swait(b):
        pltpu.make_async_copy(hbufs[b], zacc.at[didxs[b]], ssems[b]).wait()

    for p, hp in ((0, h0_hbm), (1, h1_hbm)):
        @plsc.parallel_loop(0, CH, 1)
        def zrow(r):
            for g in range(HIDDEN // 16):
                hbuf0[r, pl.ds(g * 16, 16)] = jnp.zeros((16,), jnp.float32)
                hbuf1[r, pl.ds(g * 16, 16)] = jnp.zeros((16,), jnp.float32)

        for g in range(CH // 16):
            didx1[pl.ds(g * 16, 16)] = jnp.zeros((16,), jnp.int32)

        for k in range(640 // CH):
            pltpu.sync_copy(hbuf0, zacc.at[pl.ds(s * 640 + k * CH, CH)])
        plsc.subcore_barrier()

        sissue(1)
        missue(0, 0)
        mwait(0)
        unpack(0)
        gissue(hp, 0)
        missue(1, 1)
        missue(2, 0)
        cissue(0, 0)
        cissue(1, 1)

        def pair(ii, _):
            for b in (0, 1):
                cc = 2 * ii + b
                nb = 1 - b
                gwait(hp, b)
                swait(nb)
                mwait(nb)
                unpack(nb)
                gissue(hp, nb)
                missue(cc + 3, nb)
                cwait(b)
                _scale_chunk(coefs[b], hbufs[b])
                cissue(cc + 2, b)
                sissue(b)
            return 0

        lax.fori_loop(0, (NCH - 1) // 2, pair, 0)
        gwait(hp, 0)
        swait(1)
        cwait(0)
        _scale_chunk(coefb0, hbuf0)
        sissue(0)
        swait(0)
        mwait(1)
        mwait(0)
        cwait(1)

        plsc.subcore_barrier()
        pltpu.sync_copy(zacc.at[pl.ds(s * 640, 640)],
                        zp_out.at[c, p, pl.ds(s * 640, 640)])
        plsc.subcore_barrier()


_sc_layer = functools.partial(
    pl.kernel,
    _layer_body,
    out_type=jax.ShapeDtypeStruct((2, 2, NPAD, HIDDEN), jnp.float32),
    mesh=_MESH,
    scratch_types=[
        pltpu.VMEM((CH,), jnp.int32),
        pltpu.VMEM((CH,), jnp.int32),
        pltpu.VMEM((2, CH), jnp.float32),
        pltpu.VMEM((2, CH), jnp.float32),
        pltpu.VMEM((CH,), jnp.int32),
        pltpu.VMEM((CH,), jnp.int32),
        pltpu.VMEM((CH,), jnp.int32),
        pltpu.VMEM((CH, HIDDEN), jnp.float32),
        pltpu.VMEM((CH, HIDDEN), jnp.float32),
        pltpu.VMEM_SHARED((NPAD, HIDDEN), jnp.float32),
        pltpu.SemaphoreType.DMA,
        pltpu.SemaphoreType.DMA,
        pltpu.SemaphoreType.DMA,
        pltpu.SemaphoreType.DMA,
        pltpu.SemaphoreType.DMA,
        pltpu.SemaphoreType.DMA,
        pltpu.SemaphoreType.DMA,
        pltpu.SemaphoreType.DMA,
    ],
)()


def kernel(h, edge_index, d, w_real, w_imag, t1r_W, t1r_b, t1i_W, t1i_b,
           W1_0, b1_0, W2_0, b2_0, W1_1, b1_1, W2_1, b2_1, t2_W, t2_b):
    src3d = edge_index[0].reshape(TILES, NCH, CH)
    dst3d = edge_index[1].reshape(TILES, NCH, CH)
    wr3d = w_real.reshape(TILES, NCH, CH)
    wi3d = w_imag.reshape(TILES, NCH, CH)
    d1 = d.reshape(N_NODES)

    midx, coef = _sc_coef(d1, src3d, dst3d, wr3d, wi3d)
    h0, h1 = _tc_pre(h, t1r_W, t1r_b, t1i_W, t1i_b)
    zp = _sc_layer(h0, h1, midx, coef)
    h0, h1 = _tc_mid(zp, W1_0, b1_0, W2_0, b2_0)
    zp = _sc_layer(h0, h1, midx, coef)
    return _tc_post(zp, W1_1, b1_1, W2_1, b2_1, t2_W, t2_b)[:N_NODES]
